# Initial kernel scaffold; baseline (speedup 1.0000x reference)
import functools

import jax
import jax.numpy as jnp
from jax import lax
from jax.experimental import pallas as pl
from jax.experimental.pallas import tpu as pltpu
from jax.experimental.pallas import tpu_sc as plsc

B = 4096
L = 200
D = 32
NC = 2
NS = 16
NW = NC * NS
BPW = B // NW
ROWS = BPW * L
OPB = D * L
GATHER_CHUNKS = ((0, 104), (104, 96))


def _sc_embed_body(x_hbm, table_hbm, out_hbm, idx_v, rows_v, out_v, gsem):
    wid = lax.axis_index("s") * NC + lax.axis_index("c")
    b0 = wid * BPW

    pltpu.sync_copy(x_hbm.at[pl.ds(b0 * L, ROWS)], idx_v)

    lane = lax.iota(jnp.int32, 16)
    dst0 = lane * L
    dst1 = (lane + 16) * L

    def batch_body(brel, carry):
        base = brel * L
        cps = [
            pltpu.async_copy(
                table_hbm.at[idx_v.at[pl.ds(base + off, n)]],
                rows_v.at[pl.ds(off, n)],
                gsem,
            )
            for off, n in GATHER_CHUNKS
        ]
        for cp in cps:
            cp.wait()

        def tbody(l, c):
            v0 = rows_v[l, pl.ds(0, 16)]
            v1 = rows_v[l, pl.ds(16, 16)]
            plsc.store_scatter(out_v, [dst0 + l], v0)
            plsc.store_scatter(out_v, [dst1 + l], v1)
            return c

        lax.fori_loop(0, L, tbody, 0, unroll=4)

        pltpu.sync_copy(
            out_v, out_hbm.at[pl.ds((b0 + brel) * OPB, OPB)]
        )
        return carry

    lax.fori_loop(0, BPW, batch_body, 0)


@jax.jit
def _embed(x_flat, table):
    mesh = plsc.VectorSubcoreMesh(
        core_axis_name="c", subcore_axis_name="s", num_cores=NC, num_subcores=NS
    )
    return pl.kernel(
        _sc_embed_body,
        out_type=jax.ShapeDtypeStruct((B * D * L,), jnp.float32),
        mesh=mesh,
        scratch_types=[
            pltpu.VMEM((ROWS,), jnp.int32),
            pltpu.VMEM((L, D), jnp.float32),
            pltpu.VMEM((OPB,), jnp.float32),
            pltpu.SemaphoreType.DMA,
        ],
    )(x_flat, table)


def kernel(x, table):
    out = _embed(x.reshape(-1).astype(jnp.int32), table)
    return out.reshape(B, D, L)


# speedup vs baseline: 1.3807x; 1.3807x over previous
import functools

import jax
import jax.numpy as jnp
from jax import lax
from jax.experimental import pallas as pl
from jax.experimental.pallas import tpu as pltpu
from jax.experimental.pallas import tpu_sc as plsc

B = 4096
L = 200
D = 32
NC = 2
NS = 16
NW = NC * NS
BPW = B // NW
ROWS = BPW * L
OPB = D * L
GATHER_CHUNKS = ((0, 104), (104, 96))


def _sc_embed_body(x_hbm, table_hbm, out_hbm, idx_v, rows_v, out_v, gsem):
    wid = lax.axis_index("s") * NC + lax.axis_index("c")
    b0 = wid * BPW

    pltpu.sync_copy(x_hbm.at[pl.ds(b0 * L, ROWS)], idx_v)

    lane = lax.iota(jnp.int32, 16)
    dst0 = lane * L
    dst1 = (lane + 16) * L

    def batch_body(brel, carry):
        base = brel * L
        cps = [
            pltpu.async_copy(
                table_hbm.at[idx_v.at[pl.ds(base + off, n)]],
                rows_v.at[pl.ds(off, n)],
                gsem,
            )
            for off, n in GATHER_CHUNKS
        ]
        for cp in cps:
            cp.wait()

        def tbody(l, c):
            v0 = rows_v[l, pl.ds(0, 16)]
            v1 = rows_v[l, pl.ds(16, 16)]
            plsc.store_scatter(out_v, [dst0 + l], v0)
            plsc.store_scatter(out_v, [dst1 + l], v1)
            return c

        lax.fori_loop(0, L, tbody, 0, unroll=4)

        pltpu.sync_copy(
            out_v, out_hbm.at[pl.ds((b0 + brel) * OPB, OPB)]
        )
        return carry

    lax.fori_loop(0, BPW, batch_body, 0)


@jax.jit
def _embed(x_flat, table):
    mesh = plsc.VectorSubcoreMesh(
        core_axis_name="c", subcore_axis_name="s", num_cores=NC, num_subcores=NS
    )
    return pl.kernel(
        _sc_embed_body,
        out_type=jax.ShapeDtypeStruct((B * D * L,), jnp.float32),
        mesh=mesh,
        scratch_types=[
            pltpu.VMEM((ROWS,), jnp.int32),
            pltpu.VMEM((L, D), jnp.float32),
            pltpu.VMEM((OPB,), jnp.float32),
            pltpu.SemaphoreType.DMA,
        ],
        compiler_params=pltpu.CompilerParams(
            needs_layout_passes=False, use_tc_tiling_on_sc=False
        ),
    )(x_flat, table)


def kernel(x, table):
    out = _embed(x.reshape(-1).astype(jnp.int32), table)
    return out.reshape(B, D, L)
